# Initial kernel scaffold; baseline (speedup 1.0000x reference)
import jax
import jax.numpy as jnp
from jax.experimental import pallas as pl


def kernel(embedding_user, embedding_item, edge_index):
    raise NotImplementedError("write your pallas kernel here")


# speedup vs baseline: 2.5456x; 2.5456x over previous
import functools
import jax
import jax.numpy as jnp
from jax import lax
from jax.experimental import pallas as pl
from jax.experimental.pallas import tpu as pltpu
from jax.experimental.pallas import tpu_sc as plsc

N_USERS = 25000
N_ITEMS = 25000
N = N_USERS + N_ITEMS
E = 800000
D = 64
DH = D // 2
N_LAYERS = 3

NC = 2
NS = 16
HALF = N // NC
PER_TILE = 1568
ROWS = NS * PER_TILE
TRASH = HALF
K = 128
NCHUNK = E // K
TILE_ITERS = (NCHUNK + NS - 1) // NS
W = 8

_mesh = plsc.VectorSubcoreMesh(core_axis_name="c", subcore_axis_name="s")
_sc_params = pltpu.CompilerParams(use_tc_tiling_on_sc=False)


def _fill_zero(buf, nrows, ncols):
    def row(r, _):
        for j in range(ncols // 16):
            buf[r, pl.ds(j * 16, 16)] = jnp.zeros((16,), jnp.float32)
        return 0
    lax.fori_loop(0, nrows, row, 0)


def _localize(dstv, ldstv, base):
    for j in range(K // 16):
        d = dstv[pl.ds(j * 16, 16)]
        ld = d - base
        ok = (ld >= 0) & (ld < HALF)
        ldstv[pl.ds(j * 16, 16)] = jnp.where(ok, ld, TRASH)


def _scatter_pass(idx_hbm, rows_src, agg_sh, idxv, ldstv, s, base,
                  gather_tab=None, rows_v=None, sem=None, src_hbm=None):
    def body(i, _):
        cidx = i * NS + s

        @pl.when(cidx < NCHUNK)
        def _():
            start = cidx * K
            pltpu.sync_copy(idx_hbm.at[pl.ds(start, K)], idxv)
            if gather_tab is not None:
                pltpu.sync_copy(src_hbm.at[pl.ds(start, K)], ldstv)
                pltpu.async_copy(gather_tab.at[ldstv], rows_v, sem).wait()
            _localize(idxv, ldstv, base)
            pltpu.sync_copy(rows_src if rows_v is None else rows_v,
                            agg_sh.at[ldstv], add=True)
        return 0

    lax.fori_loop(0, TILE_ITERS, body, 0)


@functools.partial(
    pl.kernel, mesh=_mesh, compiler_params=_sc_params,
    out_type=(jax.ShapeDtypeStruct((NC, ROWS, W), jnp.float32),
              jax.ShapeDtypeStruct((NC, ROWS, W), jnp.float32)),
    scratch_types=[
        pltpu.VMEM((K,), jnp.int32),
        pltpu.VMEM((K,), jnp.int32),
        pltpu.VMEM((K, W), jnp.float32),
        pltpu.VMEM((PER_TILE, W), jnp.float32),
        pltpu.VMEM_SHARED((ROWS, W), jnp.float32),
    ],
)
def _sc_degrees(src_hbm, dst_hbm, ones_hbm, zeros_hbm, dego_hbm, degi_hbm,
                idxv, ldstv, ones_v, zbuf_v, agg_sh):
    c = lax.axis_index("c")
    s = lax.axis_index("s")
    base = c * HALF
    pltpu.sync_copy(ones_hbm, ones_v)
    pltpu.sync_copy(zeros_hbm, zbuf_v)
    pltpu.sync_copy(zbuf_v, agg_sh.at[pl.ds(s * PER_TILE, PER_TILE)])
    plsc.subcore_barrier()
    _scatter_pass(src_hbm, ones_v, agg_sh, idxv, ldstv, s, base)
    plsc.subcore_barrier()
    pltpu.sync_copy(agg_sh.at[pl.ds(s * PER_TILE, PER_TILE)],
                    dego_hbm.at[c, pl.ds(s * PER_TILE, PER_TILE)])
    pltpu.sync_copy(zbuf_v, agg_sh.at[pl.ds(s * PER_TILE, PER_TILE)])
    plsc.subcore_barrier()
    _scatter_pass(dst_hbm, ones_v, agg_sh, idxv, ldstv, s, base)
    plsc.subcore_barrier()
    pltpu.sync_copy(agg_sh.at[pl.ds(s * PER_TILE, PER_TILE)],
                    degi_hbm.at[c, pl.ds(s * PER_TILE, PER_TILE)])


@functools.partial(
    pl.kernel, mesh=_mesh, compiler_params=_sc_params,
    out_type=jax.ShapeDtypeStruct((NC, 2, ROWS, DH), jnp.float32),
    scratch_types=[
        pltpu.VMEM((K,), jnp.int32),
        pltpu.VMEM((K,), jnp.int32),
        pltpu.VMEM((K, DH), jnp.float32),
        pltpu.VMEM((PER_TILE, DH), jnp.float32),
        pltpu.SemaphoreType.DMA,
        pltpu.VMEM_SHARED((ROWS, DH), jnp.float32),
    ],
)
def _sc_layer(hs_lo_hbm, hs_hi_hbm, src_hbm, dst_hbm, agg_hbm,
              dstv, ldstv, rows_v, zbuf_v, sem, agg_sh):
    c = lax.axis_index("c")
    s = lax.axis_index("s")
    base = c * HALF
    _fill_zero(zbuf_v, PER_TILE, DH)
    for p, tab in ((0, hs_lo_hbm), (1, hs_hi_hbm)):
        pltpu.sync_copy(zbuf_v, agg_sh.at[pl.ds(s * PER_TILE, PER_TILE)])
        plsc.subcore_barrier()
        _scatter_pass(dst_hbm, None, agg_sh, dstv, ldstv, s, base,
                      gather_tab=tab, rows_v=rows_v, sem=sem, src_hbm=src_hbm)
        plsc.subcore_barrier()
        pltpu.sync_copy(agg_sh.at[pl.ds(s * PER_TILE, PER_TILE)],
                        agg_hbm.at[c, p, pl.ds(s * PER_TILE, PER_TILE)])


def _tc_init_body(do_ref, di_ref, h0_ref,
                  no_ref, ni_ref, hslo_ref, hshi_ref, aclo_ref, achi_ref):
    do = do_ref[...]
    di = di_ref[...]
    h0 = h0_ref[...]
    no = jnp.where(do > 0, lax.rsqrt(jnp.maximum(do, 1e-30)), 0.0)
    ni = jnp.where(di > 0, lax.rsqrt(jnp.maximum(di, 1e-30)), 0.0)
    no_ref[...] = no
    ni_ref[...] = ni
    hslo_ref[...] = h0[:, :DH] * no
    hshi_ref[...] = h0[:, DH:] * no
    aclo_ref[...] = 0.25 * h0[:, :DH]
    achi_ref[...] = 0.25 * h0[:, DH:]


def _tc_layer_body(agg_ref, ni_ref, no_ref, acc_ref, accn_ref, hs_ref):
    h = agg_ref[...] * ni_ref[...]
    accn_ref[...] = acc_ref[...] + 0.25 * h
    hs_ref[...] = h * no_ref[...]


_RB = 1000


def _tc_init(do, di, h0):
    grid = (N // _RB,)
    col = pl.BlockSpec((_RB, 1), lambda i: (i, 0))
    mat = pl.BlockSpec((_RB, D), lambda i: (i, 0))
    half = pl.BlockSpec((_RB, DH), lambda i: (i, 0))
    return pl.pallas_call(
        _tc_init_body,
        grid=grid,
        in_specs=[col, col, mat],
        out_specs=[col, col, half, half, half, half],
        out_shape=[
            jax.ShapeDtypeStruct((N, 1), jnp.float32),
            jax.ShapeDtypeStruct((N, 1), jnp.float32),
            jax.ShapeDtypeStruct((N, DH), jnp.float32),
            jax.ShapeDtypeStruct((N, DH), jnp.float32),
            jax.ShapeDtypeStruct((N, DH), jnp.float32),
            jax.ShapeDtypeStruct((N, DH), jnp.float32),
        ],
    )(do, di, h0)


def _tc_layer(agg, ni, no, acc):
    grid = (N // _RB,)
    col = pl.BlockSpec((_RB, 1), lambda i: (i, 0))
    half = pl.BlockSpec((_RB, DH), lambda i: (i, 0))
    return pl.pallas_call(
        _tc_layer_body,
        grid=grid,
        in_specs=[half, col, col, half],
        out_specs=[half, half],
        out_shape=[
            jax.ShapeDtypeStruct((N, DH), jnp.float32),
            jax.ShapeDtypeStruct((N, DH), jnp.float32),
        ],
    )(agg, ni, no, acc)


@jax.jit
def kernel(embedding_user, embedding_item, edge_index):
    src = edge_index[0]
    dst = edge_index[1]
    h0 = jnp.concatenate([embedding_user, embedding_item], axis=0)

    ones8 = jnp.ones((K, W), jnp.float32)
    zeros8 = jnp.zeros((PER_TILE, W), jnp.float32)
    dego2, degi2 = _sc_degrees(src, dst, ones8, zeros8)
    do = dego2[:, :HALF, 0].reshape(N, 1)
    di = degi2[:, :HALF, 0].reshape(N, 1)

    no, ni, hs_lo, hs_hi, acc_lo, acc_hi = _tc_init(do, di, h0)

    def step(carry, _):
        hlo, hhi, alo, ahi = carry
        agg4 = _sc_layer(hlo, hhi, src, dst)
        agg_lo = agg4[:, 0, :HALF, :].reshape(N, DH)
        agg_hi = agg4[:, 1, :HALF, :].reshape(N, DH)
        alo_n, hlo_n = _tc_layer(agg_lo, ni, no, alo)
        ahi_n, hhi_n = _tc_layer(agg_hi, ni, no, ahi)
        return (hlo_n, hhi_n, alo_n, ahi_n), None

    (_, _, acc_lo, acc_hi), _ = lax.scan(
        step, (hs_lo, hs_hi, acc_lo, acc_hi), None, length=N_LAYERS)
    return jnp.concatenate([acc_lo, acc_hi], axis=1)


# speedup vs baseline: 3.3588x; 1.3195x over previous
import functools
import jax
import jax.numpy as jnp
from jax import lax
from jax.experimental import pallas as pl
from jax.experimental.pallas import tpu as pltpu
from jax.experimental.pallas import tpu_sc as plsc

N_USERS = 25000
N_ITEMS = 25000
N = N_USERS + N_ITEMS
E = 800000
D = 64
DH = D // 2
N_LAYERS = 3

NC = 2
NS = 16
HALF = N // NC
PER_TILE = 1568
ROWS = NS * PER_TILE
TRASH = HALF
K = 128
TILE_ITERS = (E // K + NS - 1) // NS
NCHUNK = TILE_ITERS * NS
EPAD = NCHUNK * K
W = 8

_mesh = plsc.VectorSubcoreMesh(core_axis_name="c", subcore_axis_name="s")
_sc_params = pltpu.CompilerParams(use_tc_tiling_on_sc=False)


def _fill_zero(buf, nrows, ncols):
    def row(r, _):
        for j in range(ncols // 16):
            buf[r, pl.ds(j * 16, 16)] = jnp.zeros((16,), jnp.float32)
        return 0
    lax.fori_loop(0, nrows, row, 0)


def _localize(dstv, ldstv, base):
    for j in range(K // 16):
        d = dstv[pl.ds(j * 16, 16)]
        ld = d - base
        ok = (ld >= 0) & (ld < HALF)
        ldstv[pl.ds(j * 16, 16)] = jnp.where(ok, ld, TRASH)


def _scatter_pass(idx_hbm, rows_src, agg_sh, idxv, ldstv, s, base):
    def body(i, _):
        start = (i * NS + s) * K
        pltpu.sync_copy(idx_hbm.at[pl.ds(start, K)], idxv)
        _localize(idxv, ldstv, base)
        pltpu.sync_copy(rows_src, agg_sh.at[ldstv], add=True)
        return 0

    lax.fori_loop(0, TILE_ITERS, body, 0)


@functools.partial(
    pl.kernel, mesh=_mesh, compiler_params=_sc_params,
    out_type=(jax.ShapeDtypeStruct((NC, ROWS, W), jnp.float32),
              jax.ShapeDtypeStruct((NC, ROWS, W), jnp.float32)),
    scratch_types=[
        pltpu.VMEM((K,), jnp.int32),
        pltpu.VMEM((K,), jnp.int32),
        pltpu.VMEM((K, W), jnp.float32),
        pltpu.VMEM((PER_TILE, W), jnp.float32),
        pltpu.VMEM_SHARED((ROWS, W), jnp.float32),
    ],
)
def _sc_degrees(src_hbm, dst_hbm, ones_hbm, zeros_hbm, dego_hbm, degi_hbm,
                idxv, ldstv, ones_v, zbuf_v, agg_sh):
    c = lax.axis_index("c")
    s = lax.axis_index("s")
    base = c * HALF
    pltpu.sync_copy(ones_hbm, ones_v)
    pltpu.sync_copy(zeros_hbm, zbuf_v)
    pltpu.sync_copy(zbuf_v, agg_sh.at[pl.ds(s * PER_TILE, PER_TILE)])
    plsc.subcore_barrier()
    _scatter_pass(src_hbm, ones_v, agg_sh, idxv, ldstv, s, base)
    plsc.subcore_barrier()
    pltpu.sync_copy(agg_sh.at[pl.ds(s * PER_TILE, PER_TILE)],
                    dego_hbm.at[c, pl.ds(s * PER_TILE, PER_TILE)])
    pltpu.sync_copy(zbuf_v, agg_sh.at[pl.ds(s * PER_TILE, PER_TILE)])
    plsc.subcore_barrier()
    _scatter_pass(dst_hbm, ones_v, agg_sh, idxv, ldstv, s, base)
    plsc.subcore_barrier()
    pltpu.sync_copy(agg_sh.at[pl.ds(s * PER_TILE, PER_TILE)],
                    degi_hbm.at[c, pl.ds(s * PER_TILE, PER_TILE)])


@functools.partial(
    pl.kernel, mesh=_mesh, compiler_params=_sc_params,
    out_type=jax.ShapeDtypeStruct((NC, 2, ROWS, DH), jnp.float32),
    scratch_types=[
        pltpu.VMEM((K,), jnp.int32),
        pltpu.VMEM((K,), jnp.int32),
        pltpu.VMEM((K,), jnp.int32),
        pltpu.VMEM((K,), jnp.int32),
        pltpu.VMEM((K, DH), jnp.float32),
        pltpu.VMEM((K, DH), jnp.float32),
        pltpu.VMEM((PER_TILE, DH), jnp.float32),
        pltpu.SemaphoreType.DMA,
        pltpu.SemaphoreType.DMA,
        pltpu.VMEM_SHARED((ROWS, DH), jnp.float32),
    ],
)
def _sc_layer(hs_lo_hbm, hs_hi_hbm, src_hbm, dst_hbm, agg_hbm,
              idxa, idxb, dstv, ldstv, rowsa, rowsb, zbuf_v,
              sema, semb, agg_sh):
    c = lax.axis_index("c")
    s = lax.axis_index("s")
    base = c * HALF
    _fill_zero(zbuf_v, PER_TILE, DH)
    for p, tab in ((0, hs_lo_hbm), (1, hs_hi_hbm)):
        pltpu.sync_copy(zbuf_v, agg_sh.at[pl.ds(s * PER_TILE, PER_TILE)])
        plsc.subcore_barrier()

        def issue(i, idxbuf, rowbuf, sem):
            start = (i * NS + s) * K
            pltpu.sync_copy(src_hbm.at[pl.ds(start, K)], idxbuf)
            pltpu.async_copy(tab.at[idxbuf], rowbuf, sem)

        def drain(i, idxbuf, rowbuf, sem):
            pltpu.make_async_copy(tab.at[idxbuf], rowbuf, sem).wait()
            start = (i * NS + s) * K
            pltpu.sync_copy(dst_hbm.at[pl.ds(start, K)], dstv)
            _localize(dstv, ldstv, base)
            pltpu.sync_copy(rowbuf, agg_sh.at[ldstv], add=True)

        issue(0, idxa, rowsa, sema)

        def body(j, _):
            i0 = 2 * j
            issue(i0 + 1, idxb, rowsb, semb)
            drain(i0, idxa, rowsa, sema)
            issue(i0 + 2, idxa, rowsa, sema)
            drain(i0 + 1, idxb, rowsb, semb)
            return 0

        lax.fori_loop(0, (TILE_ITERS - 1) // 2, body, 0)
        drain(TILE_ITERS - 1, idxa, rowsa, sema)

        plsc.subcore_barrier()
        pltpu.sync_copy(agg_sh.at[pl.ds(s * PER_TILE, PER_TILE)],
                        agg_hbm.at[c, p, pl.ds(s * PER_TILE, PER_TILE)])


def _tc_init_body(do_ref, di_ref, h0_ref,
                  no_ref, ni_ref, hslo_ref, hshi_ref, aclo_ref, achi_ref):
    do = do_ref[...]
    di = di_ref[...]
    h0 = h0_ref[...]
    no = jnp.where(do > 0, lax.rsqrt(jnp.maximum(do, 1e-30)), 0.0)
    ni = jnp.where(di > 0, lax.rsqrt(jnp.maximum(di, 1e-30)), 0.0)
    no_ref[...] = no
    ni_ref[...] = ni
    hslo_ref[...] = h0[:, :DH] * no
    hshi_ref[...] = h0[:, DH:] * no
    aclo_ref[...] = 0.25 * h0[:, :DH]
    achi_ref[...] = 0.25 * h0[:, DH:]


def _tc_layer_body(agg_ref, ni_ref, no_ref, acc_ref, accn_ref, hs_ref):
    h = agg_ref[...] * ni_ref[...]
    accn_ref[...] = acc_ref[...] + 0.25 * h
    hs_ref[...] = h * no_ref[...]


_RB = 1000


def _tc_init(do, di, h0):
    grid = (N // _RB,)
    col = pl.BlockSpec((_RB, 1), lambda i: (i, 0))
    mat = pl.BlockSpec((_RB, D), lambda i: (i, 0))
    half = pl.BlockSpec((_RB, DH), lambda i: (i, 0))
    return pl.pallas_call(
        _tc_init_body,
        grid=grid,
        in_specs=[col, col, mat],
        out_specs=[col, col, half, half, half, half],
        out_shape=[
            jax.ShapeDtypeStruct((N, 1), jnp.float32),
            jax.ShapeDtypeStruct((N, 1), jnp.float32),
            jax.ShapeDtypeStruct((N, DH), jnp.float32),
            jax.ShapeDtypeStruct((N, DH), jnp.float32),
            jax.ShapeDtypeStruct((N, DH), jnp.float32),
            jax.ShapeDtypeStruct((N, DH), jnp.float32),
        ],
    )(do, di, h0)


def _tc_layer(agg, ni, no, acc):
    grid = (N // _RB,)
    col = pl.BlockSpec((_RB, 1), lambda i: (i, 0))
    half = pl.BlockSpec((_RB, DH), lambda i: (i, 0))
    return pl.pallas_call(
        _tc_layer_body,
        grid=grid,
        in_specs=[half, col, col, half],
        out_specs=[half, half],
        out_shape=[
            jax.ShapeDtypeStruct((N, DH), jnp.float32),
            jax.ShapeDtypeStruct((N, DH), jnp.float32),
        ],
    )(agg, ni, no, acc)


@jax.jit
def kernel(embedding_user, embedding_item, edge_index):
    pad_trash = jnp.full((EPAD - E,), -1, jnp.int32)
    pad_zero = jnp.zeros((EPAD - E,), jnp.int32)
    src = jnp.concatenate([edge_index[0], pad_trash])
    dst = jnp.concatenate([edge_index[1], pad_trash])
    src_g = jnp.concatenate([edge_index[0], pad_zero])
    h0 = jnp.concatenate([embedding_user, embedding_item], axis=0)

    ones8 = jnp.ones((K, W), jnp.float32)
    zeros8 = jnp.zeros((PER_TILE, W), jnp.float32)
    dego2, degi2 = _sc_degrees(src, dst, ones8, zeros8)
    do = dego2[:, :HALF, 0].reshape(N, 1)
    di = degi2[:, :HALF, 0].reshape(N, 1)

    no, ni, hs_lo, hs_hi, acc_lo, acc_hi = _tc_init(do, di, h0)

    def step(carry, _):
        hlo, hhi, alo, ahi = carry
        agg4 = _sc_layer(hlo, hhi, src_g, dst)
        agg_lo = agg4[:, 0, :HALF, :].reshape(N, DH)
        agg_hi = agg4[:, 1, :HALF, :].reshape(N, DH)
        alo_n, hlo_n = _tc_layer(agg_lo, ni, no, alo)
        ahi_n, hhi_n = _tc_layer(agg_hi, ni, no, ahi)
        return (hlo_n, hhi_n, alo_n, ahi_n), None

    (_, _, acc_lo, acc_hi), _ = lax.scan(
        step, (hs_lo, hs_hi, acc_lo, acc_hi), None, length=N_LAYERS)
    return jnp.concatenate([acc_lo, acc_hi], axis=1)
